# Initial kernel scaffold; baseline (speedup 1.0000x reference)
import jax
import jax.numpy as jnp
from jax.experimental import pallas as pl


def kernel(hidden_states, Wg, bg, Wgu, bgu, Wd, bd):
    raise NotImplementedError("write your pallas kernel here")


# speedup vs baseline: 14.8710x; 14.8710x over previous
import functools

import jax
import jax.numpy as jnp
from jax.experimental import pallas as pl
from jax.experimental.pallas import tpu as pltpu

B, S, D = 1, 2048, 768
E, K, F = 8, 2, 768
ALPHA, LIMIT = 1.702, 7.0

BR = 256
NB = (S * K) // BR + E
NR = NB * BR


def _router_body(x_ref, wg_ref, bg_ref, sel_ref, w_ref):
    x = x_ref[...]
    logits = jax.lax.dot_general(
        x, wg_ref[...], (((1,), (1,)), ((), ())),
        preferred_element_type=jnp.float32)
    logits = logits + bg_ref[...]
    idx8 = jax.lax.broadcasted_iota(jnp.int32, (S, E), 1)
    m1 = jnp.max(logits, axis=1, keepdims=True)
    a1 = jnp.min(jnp.where(logits == m1, idx8, E), axis=1, keepdims=True)
    masked = jnp.where(idx8 == a1, -jnp.inf, logits)
    m2 = jnp.max(masked, axis=1, keepdims=True)
    a2 = jnp.min(jnp.where(masked == m2, idx8, E), axis=1, keepdims=True)
    w1 = jax.nn.sigmoid(m1 - m2)
    sel_ref[...] = jnp.concatenate([a1, a2], axis=1)
    w_ref[...] = jnp.concatenate([w1, 1.0 - w1], axis=1)


def _mlp_body(bexp_ref, nblk_ref, xg_ref, wgu_ref, bgu_g_ref,
              bgu_u_ref, wd_ref, bd_ref, wrow_ref, yg_ref):
    i = pl.program_id(0)

    @pl.when(i < nblk_ref[0])
    def _():
        xb = xg_ref[...]
        wgu = wgu_ref[0]
        wg_ = wgu[:, :D]
        wu_ = wgu[:, D:]
        gate = jax.lax.dot_general(
            xb, wg_, (((1,), (1,)), ((), ())),
            preferred_element_type=jnp.float32) + bgu_g_ref[0]
        up = jax.lax.dot_general(
            xb, wu_, (((1,), (1,)), ((), ())),
            preferred_element_type=jnp.float32) + bgu_u_ref[0]
        gate = jnp.minimum(gate, LIMIT)
        up = jnp.clip(up, -LIMIT, LIMIT)
        glu = gate * jax.nn.sigmoid(gate * ALPHA)
        h = (up + 1.0) * glu
        y = jax.lax.dot_general(
            h, wd_ref[0], (((1,), (1,)), ((), ())),
            preferred_element_type=jnp.float32) + bd_ref[0]
        yg_ref[...] = y * wrow_ref[0]


def kernel(hidden_states, Wg, bg, Wgu, bgu, Wd, bd):
    x = hidden_states.reshape(S, D)

    sel, w = pl.pallas_call(
        _router_body,
        out_shape=(
            jax.ShapeDtypeStruct((S, K), jnp.int32),
            jax.ShapeDtypeStruct((S, K), jnp.float32),
        ),
    )(x, Wg, bg.reshape(1, E))

    key = sel.reshape(-1)
    onehot = (key[:, None] == jnp.arange(E, dtype=jnp.int32)[None, :])
    ranks = jnp.cumsum(onehot.astype(jnp.int32), axis=0)
    counts = ranks[-1]
    rank = jnp.take_along_axis(ranks, key[:, None], axis=1)[:, 0] - 1
    padded = ((counts + BR - 1) // BR) * BR
    ends = jnp.cumsum(padded)
    offs = ends - padded
    padpos = offs[key] + rank
    nblocks = (ends[-1] // BR).astype(jnp.int32)
    bstart = jnp.arange(NB, dtype=jnp.int32) * BR
    block_expert = jnp.minimum(
        jnp.searchsorted(ends, bstart, side='right').astype(jnp.int32), E - 1)

    tok_map = jnp.zeros((NR,), jnp.int32).at[padpos].set(
        jnp.arange(S * K, dtype=jnp.int32) // K)
    w_map = jnp.zeros((NR,), jnp.float32).at[padpos].set(w.reshape(-1))

    xg = x[tok_map]

    bgu_g = bgu[:, 0::2].reshape(E, 1, F)
    bgu_u = bgu[:, 1::2].reshape(E, 1, F)

    grid_spec = pltpu.PrefetchScalarGridSpec(
        num_scalar_prefetch=2,
        grid=(NB,),
        in_specs=[
            pl.BlockSpec((BR, D), lambda i, be, nb: (i, 0)),
            pl.BlockSpec((1, F, 2 * D), lambda i, be, nb: (be[i], 0, 0)),
            pl.BlockSpec((1, 1, F), lambda i, be, nb: (be[i], 0, 0)),
            pl.BlockSpec((1, 1, F), lambda i, be, nb: (be[i], 0, 0)),
            pl.BlockSpec((1, D, F), lambda i, be, nb: (be[i], 0, 0)),
            pl.BlockSpec((1, 1, D), lambda i, be, nb: (be[i], 0, 0)),
            pl.BlockSpec((1, BR, 1), lambda i, be, nb: (i, 0, 0)),
        ],
        out_specs=pl.BlockSpec((BR, D), lambda i, be, nb: (i, 0)),
    )
    yg = pl.pallas_call(
        _mlp_body,
        grid_spec=grid_spec,
        out_shape=jax.ShapeDtypeStruct((NR, D), jnp.float32),
    )(block_expert, nblocks.reshape(1), xg, Wgu.reshape(E, F, 2 * D),
      bgu_g, bgu_u, Wd, bd.reshape(E, 1, D), w_map.reshape(NB, BR, 1))

    pos = padpos.reshape(S, K)
    out = yg[pos[:, 0]] + yg[pos[:, 1]]
    return out.reshape(B, S, D)
